# baseline (speedup 1.0000x reference)
import functools

import jax
import jax.numpy as jnp
from jax import lax
from jax.experimental import pallas as pl
from jax.experimental.pallas import tpu as pltpu
from jax.experimental.pallas import tpu_sc as plsc

_gather_fn_cache = {}


def _build_gather(n_rows, dim):
  info = plsc.get_sparse_core_info()
  nw = info.num_cores * info.num_subcores
  per_w = n_rows // nw
  chunk = per_w
  while chunk * (4 + 4 * dim) > 220_000:
    chunk //= 2
  n_chunk = per_w // chunk

  mesh = plsc.VectorSubcoreMesh(core_axis_name="c", subcore_axis_name="s")

  @functools.partial(
      pl.kernel,
      mesh=mesh,
      out_type=jax.ShapeDtypeStruct((n_rows, dim), jnp.float32),
      scratch_types=[
          pltpu.VMEM((chunk,), jnp.int32),
          pltpu.VMEM((chunk, dim), jnp.float32),
          pltpu.SemaphoreType.DMA,
      ],
      compiler_params=pltpu.CompilerParams(use_tc_tiling_on_sc=False),
  )
  def gather(idx_hbm, table_hbm, out_hbm, idx_v, rows_v, sem):
    wid = lax.axis_index("s") * info.num_cores + lax.axis_index("c")
    base = pl.multiple_of(wid * per_w, 8)

    def body(i, _):
      off = pl.multiple_of(base + i * chunk, 8)
      pltpu.sync_copy(idx_hbm.at[pl.ds(off, chunk)], idx_v)
      pltpu.async_copy(table_hbm.at[idx_v], rows_v, sem).wait()
      pltpu.sync_copy(rows_v, out_hbm.at[pl.ds(off, chunk)])
      return 0

    lax.fori_loop(0, n_chunk, body, 0)

  return gather


def kernel(indices, weight):
  b, f = indices.shape
  key = (b * f, weight.shape[1])
  if key not in _gather_fn_cache:
    _gather_fn_cache[key] = _build_gather(*key)
  idx = indices.reshape(-1).astype(jnp.int32)
  out = _gather_fn_cache[key](idx, weight)
  return out.reshape(b, f, weight.shape[1])


# speedup vs baseline: 1.0123x; 1.0123x over previous
import functools

import jax
import jax.numpy as jnp
from jax import lax
from jax.experimental import pallas as pl
from jax.experimental.pallas import tpu as pltpu
from jax.experimental.pallas import tpu_sc as plsc

_gather_fn_cache = {}


def _build_gather(n_rows, dim):
  info = plsc.get_sparse_core_info()
  nw = info.num_cores * info.num_subcores
  per_w = n_rows // nw
  nbuf = 4
  chunk = per_w
  while chunk * 4 * dim * nbuf + per_w * 4 > 490_000:
    chunk //= 2
  n_chunk = per_w // chunk

  mesh = plsc.VectorSubcoreMesh(core_axis_name="c", subcore_axis_name="s")

  @functools.partial(
      pl.kernel,
      mesh=mesh,
      out_type=jax.ShapeDtypeStruct((n_rows, dim), jnp.float32),
      scratch_types=[
          pltpu.VMEM((per_w,), jnp.int32),
          [pltpu.VMEM((chunk, dim), jnp.float32) for _ in range(nbuf)],
          pltpu.SemaphoreType.DMA,
          pltpu.SemaphoreType.DMA,
      ],
      compiler_params=pltpu.CompilerParams(use_tc_tiling_on_sc=False),
  )
  def gather(idx_hbm, table_hbm, out_hbm, idx_v, bufs, gsem, ssem):
    wid = lax.axis_index("s") * info.num_cores + lax.axis_index("c")
    base = pl.multiple_of(wid * per_w, 8)
    pltpu.sync_copy(idx_hbm.at[pl.ds(base, per_w)], idx_v)

    def start_gather(i):
      pltpu.async_copy(
          table_hbm.at[idx_v.at[pl.ds(i * chunk, chunk)]], bufs[i % nbuf], gsem
      )

    def gather_done(i):
      pltpu.make_async_copy(
          table_hbm.at[idx_v.at[pl.ds(i * chunk, chunk)]], bufs[i % nbuf], gsem
      ).wait()

    def start_store(i):
      off = pl.multiple_of(base + i * chunk, 8)
      pltpu.async_copy(bufs[i % nbuf], out_hbm.at[pl.ds(off, chunk)], ssem)

    def store_done(i):
      off = pl.multiple_of(base + i * chunk, 8)
      pltpu.make_async_copy(
          bufs[i % nbuf], out_hbm.at[pl.ds(off, chunk)], ssem
      ).wait()

    for i in range(min(nbuf, n_chunk)):
      start_gather(i)
    for i in range(n_chunk):
      gather_done(i)
      start_store(i)
      if i + nbuf < n_chunk:
        store_done(i)
        start_gather(i + nbuf)
    for i in range(max(0, n_chunk - nbuf), n_chunk):
      store_done(i)

  return gather


def kernel(indices, weight):
  b, f = indices.shape
  key = (b * f, weight.shape[1])
  if key not in _gather_fn_cache:
    _gather_fn_cache[key] = _build_gather(*key)
  idx = indices.reshape(-1).astype(jnp.int32)
  out = _gather_fn_cache[key](idx, weight)
  return out.reshape(b, f, weight.shape[1])


# speedup vs baseline: 1.3100x; 1.2941x over previous
import functools

import jax
import jax.numpy as jnp
from jax import lax
from jax.experimental import pallas as pl
from jax.experimental.pallas import tpu as pltpu
from jax.experimental.pallas import tpu_sc as plsc

_gather_fn_cache = {}


def _build_gather(b, f, dim):
  info = plsc.get_sparse_core_info()
  nw = info.num_cores * info.num_subcores
  rpw = b // nw
  nbuf = 3
  chunk = rpw
  while chunk * f * 4 * dim * nbuf + rpw * f * 4 > 480_000:
    chunk //= 2
  n_chunk = rpw // chunk

  mesh = plsc.VectorSubcoreMesh(core_axis_name="c", subcore_axis_name="s")

  @functools.partial(
      pl.kernel,
      mesh=mesh,
      out_type=jax.ShapeDtypeStruct((b, f, dim), jnp.float32),
      scratch_types=[
          pltpu.VMEM((rpw, f), jnp.int32),
          [pltpu.VMEM((chunk, f, dim), jnp.float32) for _ in range(nbuf)],
          pltpu.SemaphoreType.DMA,
          pltpu.SemaphoreType.DMA,
      ],
      compiler_params=pltpu.CompilerParams(use_tc_tiling_on_sc=False),
  )
  def gather(idx_hbm, table_hbm, out_hbm, idx_v, bufs, gsem, ssem):
    wid = lax.axis_index("s") * info.num_cores + lax.axis_index("c")
    base = pl.multiple_of(wid * rpw, 8)
    pltpu.sync_copy(idx_hbm.at[pl.ds(base, rpw)], idx_v)

    def start_gathers(i):
      buf = bufs[i % nbuf]

      def row(r, _):
        pltpu.async_copy(
            table_hbm.at[idx_v.at[i * chunk + r]], buf.at[r], gsem
        )
        return 0

      lax.fori_loop(0, chunk, row, 0)

    def gathers_done(i):
      off = pl.multiple_of(base + i * chunk, 8)
      pltpu.make_async_copy(
          out_hbm.at[pl.ds(off, chunk)], bufs[i % nbuf], gsem
      ).wait()

    def store_copy(i):
      off = pl.multiple_of(base + i * chunk, 8)
      return pltpu.make_async_copy(
          bufs[i % nbuf], out_hbm.at[pl.ds(off, chunk)], ssem
      )

    for i in range(min(nbuf, n_chunk)):
      start_gathers(i)
    for i in range(n_chunk):
      gathers_done(i)
      store_copy(i).start()
      if i + nbuf < n_chunk:
        store_copy(i).wait()
        start_gathers(i + nbuf)
    for i in range(max(0, n_chunk - nbuf), n_chunk):
      store_copy(i).wait()

  return gather


def kernel(indices, weight):
  b, f = indices.shape
  key = (b, f, weight.shape[1])
  if key not in _gather_fn_cache:
    _gather_fn_cache[key] = _build_gather(*key)
  return _gather_fn_cache[key](indices.astype(jnp.int32), weight)


# speedup vs baseline: 1.4798x; 1.1296x over previous
import functools

import jax
import jax.numpy as jnp
from jax import lax
from jax.experimental import pallas as pl
from jax.experimental.pallas import tpu as pltpu
from jax.experimental.pallas import tpu_sc as plsc

_fn_cache = {}


def _build_gather(b, f, dim):
  info = plsc.get_sparse_core_info()
  nw = info.num_cores * info.num_subcores
  rpw = b // nw
  dt_n = dim // 8
  cb = 128
  n_chunk = rpw // cb
  cf = cb * f

  mesh = plsc.VectorSubcoreMesh(core_axis_name="c", subcore_axis_name="s")

  @functools.partial(
      pl.kernel,
      mesh=mesh,
      out_type=jax.ShapeDtypeStruct((f, dt_n, b // 128, 8, 128), jnp.float32),
      scratch_types=[
          pltpu.VMEM((rpw, f), jnp.int32),
          pltpu.VMEM((cf, dim), jnp.float32),
          pltpu.VMEM((f, dt_n, 8, 128), jnp.float32),
          pltpu.SemaphoreType.DMA,
          pltpu.SemaphoreType.DMA,
      ],
      compiler_params=pltpu.CompilerParams(
          use_tc_tiling_on_sc=False, needs_layout_passes=False
      ),
  )
  def gather(idx_hbm, table_hbm, out_hbm, idx_v, gbuf, tbuf, gsem, ssem):
    wid = lax.axis_index("s") * info.num_cores + lax.axis_index("c")
    base = pl.multiple_of(wid * rpw, 8)
    bt0 = wid * (rpw // 128)
    pltpu.sync_copy(idx_hbm.at[pl.ds(base, rpw)], idx_v)

    i26 = lax.iota(jnp.int32, 16) * f

    def start_gathers(i):
      def row(r, _):
        pltpu.async_copy(
            table_hbm.at[idx_v.at[i * cb + r]],
            gbuf.at[pl.ds(r * f, f)],
            gsem,
        )
        return 0

      lax.fori_loop(0, cb, row, 0)

    def gathers_done():
      pltpu.make_async_copy(out_hbm.at[:, :, 0], tbuf, gsem).wait()

    def transpose_chunk():
      def q_iter(q, _):
        ff = q // dim
        dd = q - ff * dim
        dt = dd // 8
        dl = dd - dt * 8
        dv = jnp.full((16,), dd, jnp.int32)
        for blg in range(8):
          jv = i26 + (blg * 16 * f + ff)
          vals = plsc.load_gather(gbuf, [jv, dv])
          tbuf[ff, dt, dl, pl.ds(blg * 16, 16)] = vals
        return 0

      lax.fori_loop(0, f * dim, q_iter, 0)

    def start_stores(i):
      for ff in range(f):
        for dt in range(dt_n):
          pltpu.async_copy(
              tbuf.at[ff, dt], out_hbm.at[ff, dt, bt0 + i], ssem
          )

    def stores_done(i):
      for ff in range(f):
        for dt in range(dt_n):
          pltpu.make_async_copy(
              tbuf.at[ff, dt], out_hbm.at[ff, dt, bt0 + i], ssem
          ).wait()

    start_gathers(0)
    for i in range(n_chunk):
      gathers_done()
      transpose_chunk()
      if i + 1 < n_chunk:
        start_gathers(i + 1)
      start_stores(i)
      stores_done(i)

  return gather


def kernel(indices, weight):
  b, f = indices.shape
  v, dim = weight.shape
  key = (b, f, v, dim)
  if key not in _fn_cache:
    _fn_cache[key] = _build_gather(b, f, dim)
  out5 = _fn_cache[key](indices.astype(jnp.int32), weight)
  return jnp.transpose(out5, (2, 4, 0, 1, 3)).reshape(b, f, dim)


# speedup vs baseline: 1.4867x; 1.0047x over previous
import functools

import jax
import jax.numpy as jnp
from jax import lax
from jax.experimental import pallas as pl
from jax.experimental.pallas import tpu as pltpu
from jax.experimental.pallas import tpu_sc as plsc

_fn_cache = {}


def _build_gather(b, f, dim):
  info = plsc.get_sparse_core_info()
  nw = info.num_cores * info.num_subcores
  rpw = b // nw
  dt_n = dim // 8
  cb = 64
  n_chunk = rpw // cb
  cf = cb * f

  mesh = plsc.VectorSubcoreMesh(core_axis_name="c", subcore_axis_name="s")

  @functools.partial(
      pl.kernel,
      mesh=mesh,
      out_type=jax.ShapeDtypeStruct((f, dt_n, b // 128, 8, 128), jnp.float32),
      scratch_types=[
          pltpu.VMEM((rpw, f), jnp.int32),
          [pltpu.VMEM((cf, dim), jnp.float32) for _ in range(2)],
          pltpu.VMEM((f, dt_n, 8, 128), jnp.float32),
          pltpu.SemaphoreType.DMA,
          pltpu.SemaphoreType.DMA,
      ],
      compiler_params=pltpu.CompilerParams(
          use_tc_tiling_on_sc=False, needs_layout_passes=False
      ),
  )
  def gather(idx_hbm, table_hbm, out_hbm, idx_v, gbufs, tbuf, gsem, ssem):
    wid = lax.axis_index("s") * info.num_cores + lax.axis_index("c")
    base = pl.multiple_of(wid * rpw, 8)
    bt0 = wid * (rpw // 128)
    pltpu.sync_copy(idx_hbm.at[pl.ds(base, rpw)], idx_v)

    i26 = lax.iota(jnp.int32, 16) * f

    def row_copy(i, r):
      return pltpu.make_async_copy(
          table_hbm.at[idx_v.at[i * cb + r]],
          gbufs[i % 2].at[pl.ds(r * f, f)],
          gsem,
      )

    def start_gathers(i):
      lax.fori_loop(0, cb, lambda r, _: (row_copy(i, r).start(), 0)[1], 0)

    def gathers_done(i):
      lax.fori_loop(0, cb, lambda r, _: (row_copy(i, r).wait(), 0)[1], 0)

    def transpose_chunk(i):
      gbuf = gbufs[i % 2]
      half = (i % 2) * cb

      def f_iter(ff, _):
        for dd in range(dim):
          dv = jnp.full((16,), dd, jnp.int32)
          for blg in range(cb // 16):
            jv = i26 + (blg * 16 * f + ff)
            vals = plsc.load_gather(gbuf, [jv, dv])
            tbuf[ff, dd // 8, dd % 8, pl.ds(half + blg * 16, 16)] = vals
        return 0

      lax.fori_loop(0, f, f_iter, 0)

    def store_copies(bt):
      return [
          pltpu.make_async_copy(
              tbuf.at[ff, dt], out_hbm.at[ff, dt, bt0 + bt], ssem
          )
          for ff in range(f)
          for dt in range(dt_n)
      ]

    start_gathers(0)
    for i in range(n_chunk):
      gathers_done(i)
      if i + 1 < n_chunk:
        start_gathers(i + 1)
      if i % 2 == 0 and i > 0:
        for c in store_copies(i // 2 - 1):
          c.wait()
      transpose_chunk(i)
      if i % 2 == 1:
        for c in store_copies(i // 2):
          c.start()
    for c in store_copies(n_chunk // 2 - 1):
      c.wait()

  return gather


def kernel(indices, weight):
  b, f = indices.shape
  v, dim = weight.shape
  key = (b, f, v, dim)
  if key not in _fn_cache:
    _fn_cache[key] = _build_gather(b, f, dim)
  out5 = _fn_cache[key](indices.astype(jnp.int32), weight)
  return jnp.transpose(out5, (2, 4, 0, 1, 3)).reshape(b, f, dim)


# speedup vs baseline: 1.6756x; 1.1271x over previous
import functools

import jax
import jax.numpy as jnp
from jax import lax
from jax.experimental import pallas as pl
from jax.experimental.pallas import tpu as pltpu
from jax.experimental.pallas import tpu_sc as plsc

_fn_cache = {}


def _build_gather(b, f, dim):
  info = plsc.get_sparse_core_info()
  nw = info.num_cores * info.num_subcores
  rpw = b // nw
  dt_n = dim // 8
  cb = 64
  n_chunk = rpw // cb
  cf = cb * f

  mesh = plsc.VectorSubcoreMesh(core_axis_name="c", subcore_axis_name="s")

  @functools.partial(
      pl.kernel,
      mesh=mesh,
      out_type=jax.ShapeDtypeStruct((f, dt_n, b // 128, 8, 128), jnp.float32),
      scratch_types=[
          pltpu.VMEM((rpw, f), jnp.int32),
          [pltpu.VMEM((cf, dim), jnp.float32) for _ in range(2)],
          pltpu.VMEM((f, dt_n, 8, 128), jnp.float32),
          pltpu.SemaphoreType.DMA,
          pltpu.SemaphoreType.DMA,
      ],
      compiler_params=pltpu.CompilerParams(
          use_tc_tiling_on_sc=False, needs_layout_passes=False
      ),
  )
  def gather(idx_hbm, table_hbm, out_hbm, idx_v, gbufs, tbuf, gsem, ssem):
    wid = lax.axis_index("s") * info.num_cores + lax.axis_index("c")
    base = pl.multiple_of(wid * rpw, 8)
    bt0 = wid * (rpw // 128)
    pltpu.sync_copy(idx_hbm.at[pl.ds(base, rpw)], idx_v)

    i26 = lax.iota(jnp.int32, 16) * f

    def row_copy(i, r):
      return pltpu.make_async_copy(
          table_hbm.at[idx_v.at[i * cb + r]],
          gbufs[i % 2].at[pl.ds(r * f, f)],
          gsem,
      )

    def start_gathers(i):
      lax.fori_loop(0, cb, lambda r, _: (row_copy(i, r).start(), 0)[1], 0)

    def gathers_done(i):
      pltpu.make_async_copy(
          out_hbm.at[:, :, 0, :, pl.ds(0, cb)],
          tbuf.at[:, :, :, pl.ds(0, cb)],
          gsem,
      ).wait()

    dvs = [jnp.full((16,), dd, jnp.int32) for dd in range(dim)]

    def transpose_chunk(i):
      gbuf = gbufs[i % 2]
      half = (i % 2) * cb

      def f_iter(ff, _):
        tview = tbuf.at[ff]
        for blg in range(cb // 16):
          jv = i26 + (blg * 16 * f + ff)
          vals = [plsc.load_gather(gbuf, [jv, dvs[dd]]) for dd in range(dim)]
          for dd in range(dim):
            tview[dd // 8, dd % 8, pl.ds(half + blg * 16, 16)] = vals[dd]
        return 0

      lax.fori_loop(0, f, f_iter, 0)

    def store_copies(bt):
      return [
          pltpu.make_async_copy(
              tbuf.at[ff, dt], out_hbm.at[ff, dt, bt0 + bt], ssem
          )
          for ff in range(f)
          for dt in range(dt_n)
      ]

    start_gathers(0)
    for i in range(n_chunk):
      gathers_done(i)
      if i + 1 < n_chunk:
        start_gathers(i + 1)
      if i % 2 == 0 and i > 0:
        for c in store_copies(i // 2 - 1):
          c.wait()
      transpose_chunk(i)
      if i % 2 == 1:
        for c in store_copies(i // 2):
          c.start()
    for c in store_copies(n_chunk // 2 - 1):
      c.wait()

  return gather


def kernel(indices, weight):
  b, f = indices.shape
  v, dim = weight.shape
  key = (b, f, v, dim)
  if key not in _fn_cache:
    _fn_cache[key] = _build_gather(b, f, dim)
  out5 = _fn_cache[key](indices.astype(jnp.int32), weight)
  return jnp.transpose(out5, (2, 4, 0, 1, 3)).reshape(b, f, dim)


# speedup vs baseline: 1.6817x; 1.0036x over previous
import functools

import jax
import jax.numpy as jnp
from jax import lax
from jax.experimental import pallas as pl
from jax.experimental.pallas import tpu as pltpu
from jax.experimental.pallas import tpu_sc as plsc

_fn_cache = {}


def _build_gather(b, f, dim):
  info = plsc.get_sparse_core_info()
  nw = info.num_cores * info.num_subcores
  rpw = b // nw
  dt_n = dim // 8
  cb = 64
  n_chunk = rpw // cb
  cf = cb * f

  mesh = plsc.VectorSubcoreMesh(core_axis_name="c", subcore_axis_name="s")

  @functools.partial(
      pl.kernel,
      mesh=mesh,
      out_type=jax.ShapeDtypeStruct((f, dt_n, b // 128, 8, 128), jnp.float32),
      scratch_types=[
          pltpu.VMEM((rpw, f), jnp.int32),
          [pltpu.VMEM((cf, dim), jnp.float32) for _ in range(2)],
          pltpu.VMEM((f, dt_n, 8, 128), jnp.float32),
          pltpu.SemaphoreType.DMA,
          pltpu.SemaphoreType.DMA,
      ],
      compiler_params=pltpu.CompilerParams(
          use_tc_tiling_on_sc=False, needs_layout_passes=False
      ),
  )
  def gather(idx_hbm, table_hbm, out_hbm, idx_v, gbufs, tbuf, gsem, ssem):
    wid = lax.axis_index("s") * info.num_cores + lax.axis_index("c")
    base = pl.multiple_of(wid * rpw, 8)
    bt0 = wid * (rpw // 128)
    pltpu.sync_copy(idx_hbm.at[pl.ds(base, rpw)], idx_v)

    i26 = lax.iota(jnp.int32, 16) * f

    def row_copy(i, r):
      return pltpu.make_async_copy(
          table_hbm.at[idx_v.at[i * cb + r]],
          gbufs[i % 2].at[pl.ds(r * f, f)],
          gsem,
      )

    def start_gathers(i):
      lax.fori_loop(0, cb, lambda r, _: (row_copy(i, r).start(), 0)[1], 0)

    def gathers_done(i):
      pltpu.make_async_copy(
          out_hbm.at[:, :, 0, :, pl.ds(0, cb)],
          tbuf.at[:, :, :, pl.ds(0, cb)],
          gsem,
      ).wait()

    dvs = [jnp.full((16,), dd, jnp.int32) for dd in range(dim)]

    def transpose_chunk(i):
      gbuf = gbufs[i % 2]
      half = (i % 2) * cb

      def f_iter(ff, _):
        tview = tbuf.at[ff]
        for blg in range(cb // 16):
          jv = i26 + (blg * 16 * f + ff)
          vals = [plsc.load_gather(gbuf, [jv, dvs[dd]]) for dd in range(dim)]
          for dd in range(dim):
            tview[dd // 8, dd % 8, pl.ds(half + blg * 16, 16)] = vals[dd]
        return 0

      lax.fori_loop(0, f, f_iter, 0)

    def store_copies(bt):
      return [
          pltpu.make_async_copy(
              tbuf.at[ff, dt], out_hbm.at[ff, dt, bt0 + bt], ssem
          )
          for ff in range(f)
          for dt in range(dt_n)
      ]

    start_gathers(0)
    for i in range(n_chunk):
      gathers_done(i)
      if i + 1 < n_chunk:
        start_gathers(i + 1)
      if i % 2 == 0 and i > 0:
        pltpu.make_async_copy(out_hbm.at[:, :, 0], tbuf, ssem).wait()
      transpose_chunk(i)
      if i % 2 == 1:
        for c in store_copies(i // 2):
          c.start()
    pltpu.make_async_copy(out_hbm.at[:, :, 0], tbuf, ssem).wait()

  return gather


def kernel(indices, weight):
  b, f = indices.shape
  v, dim = weight.shape
  key = (b, f, v, dim)
  if key not in _fn_cache:
    _fn_cache[key] = _build_gather(b, f, dim)
  out5 = _fn_cache[key](indices.astype(jnp.int32), weight)
  return jnp.transpose(out5, (2, 4, 0, 1, 3)).reshape(b, f, dim)
